# Initial kernel scaffold; baseline (speedup 1.0000x reference)
import jax
import jax.numpy as jnp
from jax.experimental import pallas as pl


def kernel(x, edge_index, global_features, batch, params):
    raise NotImplementedError("write your pallas kernel here")


# speedup vs baseline: 3.9741x; 3.9741x over previous
import functools

import jax
import jax.numpy as jnp
from jax import lax
from jax.experimental import pallas as pl
from jax.experimental.pallas import tpu as pltpu
from jax.experimental.pallas import tpu_sc as plsc

N = 10000
E = 320000
D = 128
NG = 64
NC = 2
NS = 16
EPT = E // NS
BLK = 80
NITER = EPT // BLK
ACCW = 144
NHALF = N // NC
NPAD = 5120
NPT = NPAD // NS


def _lrelu(x, s):
    return jnp.maximum(x, x * s)


_GATHER_DNUMS = lax.GatherDimensionNumbers(
    offset_dims=(), collapsed_slice_dims=(0,), start_index_map=(0,))


def _lane_gather(v, idx):
    return lax.gather(v, idx[:, None], _GATHER_DNUMS, (1,),
                      mode=lax.GatherScatterMode.PROMISE_IN_BOUNDS)


def _layer_norm(x, g, b, eps=1e-5):
    mu = jnp.mean(x, axis=-1, keepdims=True)
    var = jnp.mean((x - mu) ** 2, axis=-1, keepdims=True)
    return g * (x - mu) / jnp.sqrt(var + eps) + b


def _mm2_body(x_ref, wl_ref, wr_ref, xl_ref, xr_ref):
    xb = x_ref[...]
    xl_ref[...] = jnp.dot(xb, wl_ref[...], preferred_element_type=jnp.float32)
    xr_ref[...] = jnp.dot(xb, wr_ref[...], preferred_element_type=jnp.float32)


def _mm2(x, wl, wr, blkn=2000):
    grid = (N // blkn,)
    return pl.pallas_call(
        _mm2_body,
        grid=grid,
        in_specs=[
            pl.BlockSpec((blkn, D), lambda i: (i, 0)),
            pl.BlockSpec((D, D), lambda i: (0, 0)),
            pl.BlockSpec((D, D), lambda i: (0, 0)),
        ],
        out_specs=[pl.BlockSpec((blkn, D), lambda i: (i, 0))] * 2,
        out_shape=[jax.ShapeDtypeStruct((N, D), jnp.float32)] * 2,
    )(x, wl, wr)


def _sc_edge_body(xl_hbm, xr_hbm, att_hbm, src_hbm, dst_hbm, zacc_hbm, out_hbm,
                  src_v, dst_v, sidx_v, xl_rows, xr_rows, out_rows, att_v,
                  acc_sh, sem1, sem2):
    cid = lax.axis_index("c")
    sid = lax.axis_index("s")

    pltpu.sync_copy(src_hbm.at[sid], src_v)
    pltpu.sync_copy(dst_hbm.at[sid], dst_v)
    pltpu.sync_copy(att_hbm, att_v)
    off = pl.multiple_of(sid * NPT, 8)
    pltpu.sync_copy(zacc_hbm.at[pl.ds(off, NPT)],
                    acc_sh.at[pl.ds(off, NPT)])
    plsc.subcore_barrier()

    lane = lax.iota(jnp.int32, 16)
    base = cid * NHALF

    def iter_body(j, carry):
        cp1 = pltpu.async_copy(xl_hbm.at[src_v.at[j]], xl_rows, sem1)
        cp2 = pltpu.async_copy(xr_hbm.at[dst_v.at[j]], xr_rows, sem2)
        for c in range(BLK // 16):
            d16 = dst_v[j, pl.ds(c * 16, 16)] - base
            ok = (d16 >= 0) & (d16 < NHALF)
            sidx_v[pl.ds(c * 16, 16)] = jnp.where(ok, d16, NHALF)
        cp1.wait()
        cp2.wait()

        def edge_body(e, carry2):
            acc = jnp.zeros((16,), jnp.float32)
            for c in range(D // 16):
                a = xl_rows[e, pl.ds(c * 16, 16)]
                b = xr_rows[e, pl.ds(c * 16, 16)]
                z = a + b
                acc = acc + _lrelu(z, 0.2) * att_v[pl.ds(c * 16, 16)]
            for k in (1, 2, 4, 8):
                acc = acc + _lane_gather(acc, lane ^ k)
            w = jnp.exp(acc)
            for c in range(D // 16):
                out_rows[e, pl.ds(c * 16, 16)] = w * xl_rows[e, pl.ds(c * 16, 16)]
            out_rows[e, pl.ds(D, 16)] = jnp.where(lane == 0, w, 0.0)
            return carry2

        lax.fori_loop(0, BLK, edge_body, 0, unroll=False)
        pltpu.sync_copy(out_rows, acc_sh.at[sidx_v], add=True)
        return carry

    lax.fori_loop(0, NITER, iter_body, 0, unroll=False)

    plsc.subcore_barrier()
    pltpu.sync_copy(acc_sh.at[pl.ds(off, NPT)],
                    out_hbm.at[cid, pl.ds(off, NPT)])


def _sc_edge_pass(xl, xr, att, src3, dst3, zacc):
    mesh = plsc.VectorSubcoreMesh(core_axis_name="c", subcore_axis_name="s")
    kern = functools.partial(
        pl.kernel,
        mesh=mesh,
        compiler_params=pltpu.CompilerParams(use_tc_tiling_on_sc=False),
        out_type=jax.ShapeDtypeStruct((NC, NPAD, ACCW), jnp.float32),
        scratch_types=[
            pltpu.VMEM((NITER, BLK), jnp.int32),
            pltpu.VMEM((NITER, BLK), jnp.int32),
            pltpu.VMEM((BLK,), jnp.int32),
            pltpu.VMEM((BLK, D), jnp.float32),
            pltpu.VMEM((BLK, D), jnp.float32),
            pltpu.VMEM((BLK, ACCW), jnp.float32),
            pltpu.VMEM((D,), jnp.float32),
            pltpu.VMEM_SHARED((NPAD, ACCW), jnp.float32),
            pltpu.SemaphoreType.DMA,
            pltpu.SemaphoreType.DMA,
        ],
    )(_sc_edge_body)
    return kern(xl, xr, att, src3, dst3, zacc)


def _combine_core(acc_ref, xl_ref, xr_ref, att_ref, bias_ref, g_ref, b_ref):
    accs = acc_ref[0]
    numer = accs[:, :D]
    denom = jnp.sum(accs[:, D:ACCW], axis=1, keepdims=True)
    xlb = xl_ref[...]
    xrb = xr_ref[...]
    z = xlb + xrb
    eself = jnp.dot(_lrelu(z, 0.2), att_ref[...],
                    preferred_element_type=jnp.float32)
    wself = jnp.exp(eself)
    out = (numer + wself * xlb) / (denom + wself) + bias_ref[...]
    h = _lrelu(out, 0.01)
    return _layer_norm(h, g_ref[...], b_ref[...])


def _combine_mm2_body(acc_ref, xl_ref, xr_ref, att_ref, bias_ref, g_ref, b_ref,
                      wl_ref, wr_ref, xl2_ref, xr2_ref):
    h = _combine_core(acc_ref, xl_ref, xr_ref, att_ref, bias_ref, g_ref, b_ref)
    xl2_ref[...] = jnp.dot(h, wl_ref[...], preferred_element_type=jnp.float32)
    xr2_ref[...] = jnp.dot(h, wr_ref[...], preferred_element_type=jnp.float32)


def _combine_mm2(acc, xl, xr, att_col, bias, ln_g, ln_b, wl2, wr2, blkn=1000):
    grid = (N // blkn,)
    nb = NHALF // blkn
    full = lambda shape: pl.BlockSpec(shape, lambda i: tuple(0 for _ in shape))
    return pl.pallas_call(
        _combine_mm2_body,
        grid=grid,
        in_specs=[
            pl.BlockSpec((1, blkn, ACCW), lambda i: (i // nb, i % nb, 0)),
            pl.BlockSpec((blkn, D), lambda i: (i, 0)),
            pl.BlockSpec((blkn, D), lambda i: (i, 0)),
            full((D, 1)), full((1, D)), full((1, D)), full((1, D)),
            full((D, D)), full((D, D)),
        ],
        out_specs=[pl.BlockSpec((blkn, D), lambda i: (i, 0))] * 2,
        out_shape=[jax.ShapeDtypeStruct((N, D), jnp.float32)] * 2,
    )(acc, xl, xr, att_col, bias, ln_g, ln_b, wl2, wr2)


def _final_body(acc_ref, xl_ref, xr_ref, att_ref, bias_ref, g_ref, b_ref,
                batch_ref, gf_ref, pw_ref, pb_ref,
                f1w_ref, f1b_ref, n1g_ref, n1b_ref,
                f2w_ref, f2b_ref, n2g_ref, n2b_ref,
                pgw_ref, pgb_ref, l1w_ref, l1b_ref, l2w_ref, l2b_ref,
                ow_ref, ob_ref, out_ref, pool_ref):
    i = pl.program_id(0)
    h = _combine_core(acc_ref, xl_ref, xr_ref, att_ref, bias_ref, g_ref, b_ref)
    bb = batch_ref[0, 0, :]
    onehot = (bb[:, None] == lax.broadcasted_iota(jnp.int32, (1, NG), 1)
              ).astype(jnp.float32)
    part = lax.dot_general(onehot, h, (((0,), (0,)), ((), ())),
                           preferred_element_type=jnp.float32)

    @pl.when(i == 0)
    def _():
        pool_ref[...] = jnp.zeros_like(pool_ref)

    pool_ref[...] += part

    @pl.when(i == pl.num_programs(0) - 1)
    def _():
        g = jnp.dot(gf_ref[...], pw_ref[...],
                    preferred_element_type=jnp.float32) + pb_ref[...]
        g = jnp.dot(g, f1w_ref[...], preferred_element_type=jnp.float32) + f1b_ref[...]
        g = _layer_norm(_lrelu(g, 0.01), n1g_ref[...], n1b_ref[...])
        g = jnp.dot(g, f2w_ref[...], preferred_element_type=jnp.float32) + f2b_ref[...]
        g = _layer_norm(_lrelu(g, 0.01), n2g_ref[...], n2b_ref[...])
        pg = jnp.dot(g, pgw_ref[...], preferred_element_type=jnp.float32) + pgb_ref[...]
        a = pool_ref[...] + pg
        a = _lrelu(jnp.dot(a, l1w_ref[...], preferred_element_type=jnp.float32)
                   + l1b_ref[...], 0.01)
        a = _lrelu(jnp.dot(a, l2w_ref[...], preferred_element_type=jnp.float32)
                   + l2b_ref[...], 0.01)
        out_ref[...] = jnp.dot(a, ow_ref[...],
                               preferred_element_type=jnp.float32) + ob_ref[...]


def _final(acc, xl, xr, att_col, bias, ln_g, ln_b, batch3, gf, mlp, pg, fcs,
           outp, blkn=1000):
    grid = (N // blkn,)
    nb = NHALF // blkn
    full = lambda shape: pl.BlockSpec(shape, lambda i: tuple(0 for _ in shape))
    in_specs = [
        pl.BlockSpec((1, blkn, ACCW), lambda i: (i // nb, i % nb, 0)),
        pl.BlockSpec((blkn, D), lambda i: (i, 0)),
        pl.BlockSpec((blkn, D), lambda i: (i, 0)),
        full((D, 1)), full((1, D)), full((1, D)), full((1, D)),
        pl.BlockSpec((1, 1, blkn), lambda i: (i, 0, 0)),
        full((NG, 32)), full((32, D)), full((1, D)),
        full((D, D)), full((1, D)), full((1, D)), full((1, D)),
        full((D, D)), full((1, D)), full((1, D)), full((1, D)),
        full((D, D)), full((1, D)),
        full((D, D)), full((1, D)), full((D, D)), full((1, D)),
        full((D, 1)), full((1, 1)),
    ]
    args = [acc, xl, xr, att_col, bias, ln_g, ln_b, batch3, gf,
            mlp['proj_W'], mlp['proj_b'].reshape(1, D),
            mlp['fcs'][0]['W'], mlp['fcs'][0]['b'].reshape(1, D),
            mlp['norms'][0]['g'].reshape(1, D), mlp['norms'][0]['b'].reshape(1, D),
            mlp['fcs'][1]['W'], mlp['fcs'][1]['b'].reshape(1, D),
            mlp['norms'][1]['g'].reshape(1, D), mlp['norms'][1]['b'].reshape(1, D),
            pg['W'], pg['b'].reshape(1, D),
            fcs[0]['W'], fcs[0]['b'].reshape(1, D),
            fcs[1]['W'], fcs[1]['b'].reshape(1, D),
            outp['W'], outp['b'].reshape(1, 1)]
    return pl.pallas_call(
        _final_body,
        grid=grid,
        in_specs=in_specs,
        out_specs=pl.BlockSpec((NG, 1), lambda i: (0, 0)),
        out_shape=jax.ShapeDtypeStruct((NG, 1), jnp.float32),
        scratch_shapes=[pltpu.VMEM((NG, D), jnp.float32)],
    )(*args)


def kernel(x, edge_index, global_features, batch, params):
    src3 = edge_index[0].reshape(NS, NITER, BLK)
    dst3 = edge_index[1].reshape(NS, NITER, BLK)
    batch3 = batch.reshape(N // 1000, 1, 1000)
    zacc = jnp.zeros((NPAD, ACCW), jnp.float32)

    c0, c1 = params['convs']
    xl1, xr1 = _mm2(x, c0['Wl'], c0['Wr'])
    acc1 = _sc_edge_pass(xl1, xr1, c0['att'], src3, dst3, zacc)
    xl2, xr2 = _combine_mm2(acc1, xl1, xr1, c0['att'].reshape(D, 1),
                            c0['bias'].reshape(1, D), c0['ln_g'].reshape(1, D),
                            c0['ln_b'].reshape(1, D), c1['Wl'], c1['Wr'])
    acc2 = _sc_edge_pass(xl2, xr2, c1['att'], src3, dst3, zacc)
    return _final(acc2, xl2, xr2, c1['att'].reshape(D, 1),
                  c1['bias'].reshape(1, D), c1['ln_g'].reshape(1, D),
                  c1['ln_b'].reshape(1, D), batch3, global_features,
                  params['mlp'], params['proj_global'], params['fcs'],
                  params['out'])
